# Initial kernel scaffold; baseline (speedup 1.0000x reference)
import jax
import jax.numpy as jnp
from jax.experimental import pallas as pl


def kernel(inputs, Wr, br, W1, b1, W2, b2):
    raise NotImplementedError("write your pallas kernel here")


# speedup vs baseline: 4.1966x; 4.1966x over previous
import jax
import jax.numpy as jnp
from jax.experimental import pallas as pl
from jax.experimental.pallas import tpu as pltpu


def _moe_block_kernel(x_ref, wr_ref, br_ref, w1_ref, b1_ref, w2_ref, b2_ref,
                      out_ref, wout_ref, *, bt, e_dim):
    x = x_ref[...]
    xb = x.astype(jnp.bfloat16)

    logits = jnp.dot(xb, wr_ref[...].astype(jnp.bfloat16),
                     preferred_element_type=jnp.float32) + br_ref[...]
    m = jnp.max(logits, axis=-1, keepdims=True)
    ex = jnp.exp(logits - m)
    w = ex / jnp.sum(ex, axis=-1, keepdims=True)
    wout_ref[...] = w

    e_iota = jax.lax.broadcasted_iota(jnp.int32, (bt, e_dim), 1)
    idx1 = jnp.argmax(w, axis=-1)[:, None]
    oh1 = e_iota == idx1
    w_rest = jnp.where(oh1, -jnp.inf, w)
    idx2 = jnp.argmax(w_rest, axis=-1)[:, None]
    oh2 = e_iota == idx2
    mw = jnp.where(oh1 | oh2, w, 0.0)
    mw = mw / (jnp.sum(mw, axis=-1, keepdims=True) + 1e-9)

    b1 = b1_ref[...]
    b2 = b2_ref[...]
    acc = jnp.zeros(out_ref.shape, jnp.float32)
    for e in range(e_dim):
        h = jnp.dot(xb, w1_ref[e], preferred_element_type=jnp.float32)
        h = h + b1[e][None, :]
        h = 0.5 * h * (1.0 + jax.lax.erf(h * 0.7071067811865476))
        o = jnp.dot(h.astype(jnp.bfloat16), w2_ref[e],
                    preferred_element_type=jnp.float32)
        o = o + b2[e][None, :]
        acc = acc + mw[:, e][:, None] * o
    out_ref[...] = acc


def kernel(inputs, Wr, br, W1, b1, W2, b2):
    B, H = inputs.shape
    E = Wr.shape[1]
    BT = 256 if B % 256 == 0 else B

    w1b = W1.astype(jnp.bfloat16)
    w2b = W2.astype(jnp.bfloat16)
    br2 = br.reshape(1, E)

    import functools
    body = functools.partial(_moe_block_kernel, bt=BT, e_dim=E)

    combined, weights = pl.pallas_call(
        body,
        grid=(B // BT,),
        in_specs=[
            pl.BlockSpec((BT, H), lambda i: (i, 0)),
            pl.BlockSpec((H, E), lambda i: (0, 0)),
            pl.BlockSpec((1, E), lambda i: (0, 0)),
            pl.BlockSpec((E, H, H), lambda i: (0, 0, 0)),
            pl.BlockSpec((E, H), lambda i: (0, 0)),
            pl.BlockSpec((E, H, H), lambda i: (0, 0, 0)),
            pl.BlockSpec((E, H), lambda i: (0, 0)),
        ],
        out_specs=[
            pl.BlockSpec((BT, H), lambda i: (i, 0)),
            pl.BlockSpec((BT, E), lambda i: (i, 0)),
        ],
        out_shape=[
            jax.ShapeDtypeStruct((B, H), jnp.float32),
            jax.ShapeDtypeStruct((B, E), jnp.float32),
        ],
        compiler_params=pltpu.CompilerParams(
            dimension_semantics=("arbitrary",),
        ),
    )(inputs, Wr, br2, w1b, b1, w2b, b2)
    return (combined, weights)
